# speedup vs baseline: 1.0309x; 1.0309x over previous
import functools

import jax
import jax.numpy as jnp
from jax.experimental import pallas as pl
from jax.experimental.pallas import tpu as pltpu

D = 4096
H = 2048
E = 64
TOP_K = 8

BM = 512
BK = 2048
NK = D // BK


def _router_kernel(x_ref, w1_ref, b1_ref, w2_ref, b2_ref,
                   probs_ref, idx_ref, rw_ref, h_ref):
    k = pl.program_id(1)

    part = jnp.dot(x_ref[...], w1_ref[...], preferred_element_type=jnp.float32)

    @pl.when(k == 0)
    def _():
        h_ref[...] = part

    @pl.when(k > 0)
    def _():
        h_ref[...] += part

    @pl.when(k == NK - 1)
    def _():
        h = jnp.maximum(h_ref[...] + b1_ref[...], 0.0)
        logits = jnp.dot(h, w2_ref[...], preferred_element_type=jnp.float32)
        logits = logits + b2_ref[...]
        m = jnp.max(logits, axis=-1, keepdims=True)
        e = jnp.exp(logits - m)
        p = e / jnp.sum(e, axis=-1, keepdims=True)
        probs_ref[...] = p

        lanes = jax.lax.broadcasted_iota(jnp.int32, (BM, E), 1)
        work = p
        vals = []
        idxs = []
        for _ in range(TOP_K):
            mx = jnp.max(work, axis=-1, keepdims=True)
            cand = jnp.where(work == mx, lanes, E)
            ix = jnp.min(cand, axis=-1, keepdims=True)
            vals.append(mx)
            idxs.append(ix)
            work = jnp.where(lanes == ix, -1.0, work)
        tkv = jnp.concatenate(vals, axis=-1)
        tki = jnp.concatenate(idxs, axis=-1)
        idx_ref[...] = tki
        m2 = jnp.max(tkv, axis=-1, keepdims=True)
        e2 = jnp.exp(tkv - m2)
        rw_ref[...] = e2 / jnp.sum(e2, axis=-1, keepdims=True)


@functools.partial(jax.jit, static_argnames=())
def kernel(features, W1, b1, W2, b2):
    B, S, _ = features.shape
    M = B * S
    x = features.reshape(M, D)
    b1r = b1.reshape(1, H)
    b2r = b2.reshape(1, E)

    grid = (M // BM, NK)
    probs, idx, rw = pl.pallas_call(
        _router_kernel,
        grid=grid,
        in_specs=[
            pl.BlockSpec((BM, BK), lambda m, k: (m, k)),
            pl.BlockSpec((BK, H), lambda m, k: (k, 0)),
            pl.BlockSpec((1, H), lambda m, k: (0, 0)),
            pl.BlockSpec((H, E), lambda m, k: (0, 0)),
            pl.BlockSpec((1, E), lambda m, k: (0, 0)),
        ],
        out_specs=[
            pl.BlockSpec((BM, E), lambda m, k: (m, 0)),
            pl.BlockSpec((BM, TOP_K), lambda m, k: (m, 0)),
            pl.BlockSpec((BM, TOP_K), lambda m, k: (m, 0)),
        ],
        out_shape=[
            jax.ShapeDtypeStruct((M, E), jnp.float32),
            jax.ShapeDtypeStruct((M, TOP_K), jnp.int32),
            jax.ShapeDtypeStruct((M, TOP_K), jnp.float32),
        ],
        scratch_shapes=[pltpu.VMEM((BM, H), jnp.float32)],
        compiler_params=pltpu.CompilerParams(
            dimension_semantics=("parallel", "arbitrary"),
        ),
    )(x, W1, b1r, W2, b2r)

    return (probs.reshape(B, S, E),
            idx.reshape(B, S, TOP_K),
            rw.reshape(B, S, TOP_K))


# speedup vs baseline: 1.3605x; 1.3198x over previous
import functools

import jax
import jax.numpy as jnp
from jax.experimental import pallas as pl
from jax.experimental.pallas import tpu as pltpu

D = 4096
H = 2048
E = 64
TOP_K = 8

BM = 512


def _router_kernel(x_ref, w1_ref, b1_ref, w2_ref, b2_ref,
                   probs_ref, idx_ref, rw_ref):
    acc = jnp.dot(x_ref[...], w1_ref[...], preferred_element_type=jnp.float32)
    h = jnp.maximum(acc + b1_ref[...], 0.0)
    logits = jnp.dot(h, w2_ref[...], preferred_element_type=jnp.float32)
    logits = logits + b2_ref[...]

    m = jnp.max(logits, axis=-1, keepdims=True)
    e = jnp.exp(logits - m)
    p = e / jnp.sum(e, axis=-1, keepdims=True)
    probs_ref[...] = p

    lanes = jax.lax.broadcasted_iota(jnp.int32, (BM, E), 1)
    work = p
    vals = []
    idxs = []
    for _ in range(TOP_K):
        mx = jnp.max(work, axis=-1, keepdims=True)
        cand = jnp.where(work == mx, lanes, E)
        ix = jnp.min(cand, axis=-1, keepdims=True)
        vals.append(mx)
        idxs.append(ix)
        work = jnp.where(lanes == ix, -1.0, work)
    tkv = jnp.concatenate(vals, axis=-1)
    tki = jnp.concatenate(idxs, axis=-1)
    idx_ref[...] = tki
    m2 = jnp.max(tkv, axis=-1, keepdims=True)
    e2 = jnp.exp(tkv - m2)
    rw_ref[...] = e2 / jnp.sum(e2, axis=-1, keepdims=True)


@functools.partial(jax.jit, static_argnames=())
def kernel(features, W1, b1, W2, b2):
    B, S, _ = features.shape
    M = B * S
    x = features.reshape(M, D)
    b1r = b1.reshape(1, H)
    b2r = b2.reshape(1, E)

    grid = (M // BM,)
    probs, idx, rw = pl.pallas_call(
        _router_kernel,
        grid=grid,
        in_specs=[
            pl.BlockSpec((BM, D), lambda m: (m, 0)),
            pl.BlockSpec((D, H), lambda m: (0, 0)),
            pl.BlockSpec((1, H), lambda m: (0, 0)),
            pl.BlockSpec((H, E), lambda m: (0, 0)),
            pl.BlockSpec((1, E), lambda m: (0, 0)),
        ],
        out_specs=[
            pl.BlockSpec((BM, E), lambda m: (m, 0)),
            pl.BlockSpec((BM, TOP_K), lambda m: (m, 0)),
            pl.BlockSpec((BM, TOP_K), lambda m: (m, 0)),
        ],
        out_shape=[
            jax.ShapeDtypeStruct((M, E), jnp.float32),
            jax.ShapeDtypeStruct((M, TOP_K), jnp.int32),
            jax.ShapeDtypeStruct((M, TOP_K), jnp.float32),
        ],
        compiler_params=pltpu.CompilerParams(
            dimension_semantics=("arbitrary",),
        ),
    )(x, W1, b1r, W2, b2r)

    return (probs.reshape(B, S, E),
            idx.reshape(B, S, TOP_K),
            rw.reshape(B, S, TOP_K))
